# baseline (speedup 1.0000x reference)
import functools

import jax
import jax.numpy as jnp
from jax import lax
from jax.experimental import pallas as pl
from jax.experimental.pallas import tpu as pltpu
from jax.experimental.pallas import tpu_sc as plsc

_D = 128
_B = 16384

_info = plsc.get_sparse_core_info()
_NS = _info.num_subcores
_N0 = 504
_N1 = 520

_mesh = plsc.VectorSubcoreMesh(core_axis_name="c", subcore_axis_name="s")


@functools.partial(
    pl.kernel,
    mesh=_mesh,
    out_type=jax.ShapeDtypeStruct((_B, _D), jnp.float32),
    scratch_types=[
        pltpu.VMEM((_N1,), jnp.int32),
        pltpu.VMEM((_N1, _D), jnp.float32),
        pltpu.SemaphoreType.DMA,
    ],
)
def _emb_gather(idx_hbm, table_hbm, out_hbm, idx_v, rows_v, sem):
    c = lax.axis_index("c")
    s = lax.axis_index("s")

    def _gather(base, n):
        pltpu.sync_copy(idx_hbm.at[pl.ds(base, n)], idx_v.at[pl.ds(0, n)])
        pltpu.async_copy(
            table_hbm.at[idx_v.at[pl.ds(0, n)]], rows_v.at[pl.ds(0, n)], sem
        ).wait()
        pltpu.sync_copy(rows_v.at[pl.ds(0, n)], out_hbm.at[pl.ds(base, n)])

    @pl.when(c == 0)
    def _():
        _gather(s * _N0, _N0)

    @pl.when(c == 1)
    def _():
        _gather(_NS * _N0 + s * _N1, _N1)


def kernel(titles, embedding_table):
    return _emb_gather(titles.astype(jnp.int32), embedding_table)
